# Initial kernel scaffold; baseline (speedup 1.0000x reference)
import jax
import jax.numpy as jnp
from jax.experimental import pallas as pl


def kernel(A_indices, A_values, X, W, b):
    raise NotImplementedError("write your pallas kernel here")


# speedup vs baseline: 4.5295x; 4.5295x over previous
import functools

import jax
import jax.numpy as jnp
from jax import lax
from jax.experimental import pallas as pl
from jax.experimental.pallas import tpu as pltpu
from jax.experimental.pallas import tpu_sc as plsc

N = 10000
E = 320000
D = 128

NC = 2
NS = 16
NW = NC * NS
EPW = E // NW
C = 80
NCHUNK = EPW // C
NP = 10240
RPT = NP // NS
ZR = 128
LG = D // 16


def _sc_body(dst_hbm, src_hbm, val_hbm, x_hbm, out_hbm,
             dst_v, src_v, val_v, rows_v, zbuf, hsh, sem):
    c = lax.axis_index("c")
    s = lax.axis_index("s")
    w = c * NS + s

    def zrow(i, _):
        for j in range(LG):
            zbuf[i, pl.ds(j * 16, 16)] = jnp.zeros((16,), jnp.float32)
        return 0
    lax.fori_loop(0, ZR, zrow, 0)
    for k in range(RPT // ZR):
        pltpu.sync_copy(zbuf, hsh.at[pl.ds(s * RPT + k * ZR, ZR)])
    plsc.subcore_barrier()

    def chunk(t, _):
        base = w * EPW + t * C
        pltpu.sync_copy(dst_hbm.at[pl.ds(base, C)], dst_v)
        pltpu.sync_copy(src_hbm.at[pl.ds(base, C)], src_v)
        pltpu.sync_copy(val_hbm.at[pl.ds(base, C)], val_v)
        pltpu.async_copy(x_hbm.at[src_v], rows_v, sem).wait()

        def scale(eb, _):
            v16 = val_v[pl.ds(eb * 16, 16)]
            for i in range(16):
                e = eb * 16 + i
                sp = jnp.full((16,), v16[i], jnp.float32)
                for j in range(LG):
                    sl = pl.ds(j * 16, 16)
                    rows_v[e, sl] = rows_v[e, sl] * sp
            return 0
        lax.fori_loop(0, C // 16, scale, 0)

        pltpu.sync_copy(rows_v, hsh.at[dst_v], add=True)
        return 0
    lax.fori_loop(0, NCHUNK, chunk, 0)

    plsc.subcore_barrier()
    pltpu.sync_copy(hsh.at[pl.ds(s * RPT, RPT)],
                    out_hbm.at[c, pl.ds(s * RPT, RPT)])


def _sc_scatter(dst, src, val, x):
    mesh = plsc.VectorSubcoreMesh(core_axis_name="c", subcore_axis_name="s")
    f = pl.kernel(
        _sc_body,
        out_type=jax.ShapeDtypeStruct((NC, NP, D), jnp.float32),
        mesh=mesh,
        scratch_types=[
            pltpu.VMEM((C,), jnp.int32),
            pltpu.VMEM((C,), jnp.int32),
            pltpu.VMEM((C,), jnp.float32),
            pltpu.VMEM((C, D), jnp.float32),
            pltpu.VMEM((ZR, D), jnp.float32),
            pltpu.VMEM_SHARED((NP, D), jnp.float32),
            pltpu.SemaphoreType.DMA,
        ],
    )
    return f(dst, src, val, x)


def _tc_body(hp_ref, wt_ref, b_ref, o_ref):
    h = hp_ref[0] + hp_ref[1]
    y = jnp.dot(h, wt_ref[...], preferred_element_type=jnp.float32)
    o_ref[...] = jnp.maximum(y + b_ref[...], 0.0)


def _tc_linear(partials, wt, b):
    R = 2048
    grid = (NP // R,)
    return pl.pallas_call(
        _tc_body,
        grid=grid,
        in_specs=[
            pl.BlockSpec((NC, R, D), lambda i: (0, i, 0)),
            pl.BlockSpec((D, D), lambda i: (0, 0)),
            pl.BlockSpec((1, D), lambda i: (0, 0)),
        ],
        out_specs=pl.BlockSpec((R, D), lambda i: (i, 0)),
        out_shape=jax.ShapeDtypeStruct((NP, D), jnp.float32),
    )(partials, wt, b)


def kernel(A_indices, A_values, X, W, b):
    dst = A_indices[0]
    src = A_indices[1]
    partials = _sc_scatter(dst, src, A_values, X)
    return _tc_linear(partials, W.T, b.reshape(1, D))[:N]
